# baseline (speedup 1.0000x reference)
import functools

import jax
import jax.numpy as jnp
from jax import lax
from jax.experimental import pallas as pl
from jax.experimental.pallas import tpu as pltpu
from jax.experimental.pallas import tpu_sc as plsc

N = 10000
E = 320000
ED = 2 * E
EMB = 32
GE = 64
W128 = 2 * GE
NG = 128

NC = 2
NS = 16
NW = NC * NS

NP = 10240
NPW = NP // NW
NBC = 4
NBW = NPW // NBC
CH = 128
NCH = 160
PE = NW * NCH * CH
EPAD = NP - 1
RPT = NP // NS
RBL = 128
RB = RPT // RBL

_f32 = jnp.float32
_i32 = jnp.int32

_MESH = plsc.VectorSubcoreMesh(core_axis_name="c", subcore_axis_name="s")


def _tc0_body(op_t, svc_t, st_t, w1, opw, svcw, stw):
    w = w1[...]
    z = jnp.zeros((1, GE), _f32)
    opw[:, 0:GE] = jnp.dot(op_t[...], w[0:EMB, :], preferred_element_type=_f32)
    opw[:, GE:W128] = jnp.broadcast_to(z, (1000, GE))
    svcw[:, 0:GE] = jnp.dot(svc_t[...], w[EMB:2 * EMB, :],
                            preferred_element_type=_f32)
    svcw[:, GE:W128] = jnp.broadcast_to(z, (200, GE))
    stw[:, 0:GE] = jnp.dot(st_t[...], w[2 * EMB:3 * EMB, :],
                           preferred_element_type=_f32)
    stw[:, GE:W128] = jnp.broadcast_to(z, (16, GE))


def _tc0(op_t, svc_t, st_t, w1):
    return pl.pallas_call(
        _tc0_body,
        out_shape=[
            jax.ShapeDtypeStruct((1000, W128), _f32),
            jax.ShapeDtypeStruct((200, W128), _f32),
            jax.ShapeDtypeStruct((16, W128), _f32),
        ],
    )(op_t, svc_t, st_t, w1)


@functools.partial(
    pl.kernel,
    mesh=_MESH,
    out_type=[jax.ShapeDtypeStruct((NC * NP, W128), _f32)],
    scratch_types=[
        pltpu.VMEM((CH,), _i32),
        pltpu.VMEM((CH,), _i32),
        pltpu.VMEM((CH, W128), _f32),
        pltpu.VMEM((RBL, W128), _f32),
        pltpu.VMEM_SHARED((NP, W128), _f32),
        pltpu.SemaphoreType.DMA,
        pltpu.SemaphoreType.DMA,
    ],
)
def _sc_deg(dst_hbm, ones_hbm, z_hbm, deg_out, dstia, dstib, onesb, zb,
            acc_sp, sema, semb):
    c = lax.axis_index("c")
    s = lax.axis_index("s")
    wid = c * NS + s

    pltpu.sync_copy(z_hbm, zb)
    for p in range(RB):
        pltpu.sync_copy(zb, acc_sp.at[pl.ds(s * RPT + p * RBL, RBL)])
    pltpu.sync_copy(ones_hbm, onesb)
    plsc.subcore_barrier()

    def body(m, carry):
        base = (2 * m * NW + wid) * CH
        pltpu.sync_copy(dst_hbm.at[pl.ds(base, CH)], dstia)
        pltpu.async_copy(onesb, acc_sp.at[dstia], sema, add=True)
        pltpu.sync_copy(dst_hbm.at[pl.ds(base + NW * CH, CH)], dstib)
        pltpu.async_copy(onesb, acc_sp.at[dstib], semb, add=True)
        pltpu.make_async_copy(onesb, acc_sp.at[dstia], sema).wait()
        pltpu.make_async_copy(onesb, acc_sp.at[dstib], semb).wait()
        return carry

    lax.fori_loop(0, NCH // 2, body, 0)
    plsc.subcore_barrier()

    for p in range(RB):
        off = s * RPT + p * RBL
        pltpu.sync_copy(acc_sp.at[pl.ds(off, RBL)], zb)
        pltpu.sync_copy(zb, deg_out.at[pl.ds(c * NP + off, RBL)])


@functools.partial(
    pl.kernel,
    mesh=_MESH,
    out_type=[jax.ShapeDtypeStruct((NP, W128), _f32)],
    scratch_types=[
        pltpu.VMEM((NBW,), _i32),
        pltpu.VMEM((NBW,), _i32),
        pltpu.VMEM((NBW,), _i32),
        pltpu.VMEM((NBW, W128), _f32),
        pltpu.VMEM((NBW, W128), _f32),
        pltpu.VMEM((NBW, W128), _f32),
    ],
)
def _sc_embed(opi_hbm, svci_hbm, sti_hbm, opw_hbm, svcw_hbm, stw_hbm,
              h1pre_out, opi, svci, sti, bufa, bufb, bufc):
    c = lax.axis_index("c")
    s = lax.axis_index("s")
    wid = c * NS + s

    for k in range(NBC):
        off = wid * NPW + k * NBW
        sl = pl.ds(off, NBW)
        pltpu.sync_copy(opi_hbm.at[sl], opi)
        pltpu.sync_copy(svci_hbm.at[sl], svci)
        pltpu.sync_copy(sti_hbm.at[sl], sti)
        pltpu.sync_copy(opw_hbm.at[opi], bufa)
        pltpu.sync_copy(svcw_hbm.at[svci], bufb)
        pltpu.sync_copy(stw_hbm.at[sti], bufc)

        def add_body(r, carry):
            for q in range(W128 // 16):
                cs = pl.ds(q * 16, 16)
                bufa[r, cs] = bufa[r, cs] + bufb[r, cs] + bufc[r, cs]
            return carry

        lax.fori_loop(0, NBW, add_body, 0)
        pltpu.sync_copy(bufa, h1pre_out.at[sl])


@functools.partial(
    pl.kernel,
    mesh=_MESH,
    out_type=[jax.ShapeDtypeStruct((NC * NP, W128), _f32)],
    scratch_types=[
        pltpu.VMEM((CH,), _i32),
        pltpu.VMEM((CH,), _i32),
        pltpu.VMEM((CH,), _i32),
        pltpu.VMEM((CH,), _i32),
        pltpu.VMEM((CH, W128), _f32),
        pltpu.VMEM((CH, W128), _f32),
        pltpu.VMEM_SHARED((NP, W128), _f32),
        pltpu.SemaphoreType.DMA,
        pltpu.SemaphoreType.DMA,
    ],
)
def _sc_agg(src_hbm, dst_hbm, h_hbm, z_hbm, agg_out, srcia, srcib, dstia,
            dstib, rowsa, rowsb, acc_sp, sema, semb):
    c = lax.axis_index("c")
    s = lax.axis_index("s")
    wid = c * NS + s

    pltpu.sync_copy(z_hbm, rowsa)
    for p in range(RB):
        pltpu.sync_copy(rowsa, acc_sp.at[pl.ds(s * RPT + p * RBL, RBL)])
    plsc.subcore_barrier()

    pltpu.sync_copy(src_hbm.at[pl.ds(wid * CH, CH)], srcia)
    pltpu.sync_copy(dst_hbm.at[pl.ds(wid * CH, CH)], dstia)
    pltpu.async_copy(h_hbm.at[srcia], rowsa, sema)

    def body(m, carry):
        baseb = ((2 * m + 1) * NW + wid) * CH
        pltpu.sync_copy(src_hbm.at[pl.ds(baseb, CH)], srcib)
        pltpu.sync_copy(dst_hbm.at[pl.ds(baseb, CH)], dstib)
        pltpu.async_copy(h_hbm.at[srcib], rowsb, semb)
        pltpu.make_async_copy(h_hbm.at[srcia], rowsa, sema).wait()
        pltpu.sync_copy(rowsa, acc_sp.at[dstia], add=True)
        basea = ((2 * m + 2) * NW + wid) * CH
        pltpu.sync_copy(src_hbm.at[pl.ds(basea, CH)], srcia)
        pltpu.sync_copy(dst_hbm.at[pl.ds(basea, CH)], dstia)
        pltpu.async_copy(h_hbm.at[srcia], rowsa, sema)
        pltpu.make_async_copy(h_hbm.at[srcib], rowsb, semb).wait()
        pltpu.sync_copy(rowsb, acc_sp.at[dstib], add=True)
        return carry

    lax.fori_loop(0, NCH // 2, body, 0)
    pltpu.make_async_copy(h_hbm.at[srcia], rowsa, sema).wait()
    plsc.subcore_barrier()

    for p in range(RB):
        off = s * RPT + p * RBL
        pltpu.sync_copy(acc_sp.at[pl.ds(off, RBL)], rowsa)
        pltpu.sync_copy(rowsa, agg_out.at[pl.ds(c * NP + off, RBL)])


def _dis_from(d0, d1):
    deg = d0[:, 0:1] + d1[:, 0:1]
    return jnp.where(deg > 0.0, lax.rsqrt(jnp.maximum(deg, 1.0)), 0.0)


def _tc1_body(d0, d1, h1pre, h1s):
    h1s[...] = h1pre[...] * _dis_from(d0[...], d1[...])


def _tc1(d0, d1, h1pre):
    blk = NP // 80
    return pl.pallas_call(
        _tc1_body,
        grid=(80,),
        in_specs=[
            pl.BlockSpec((blk, W128), lambda j: (j, 0)),
            pl.BlockSpec((blk, W128), lambda j: (j, 0)),
            pl.BlockSpec((blk, W128), lambda j: (j, 0)),
        ],
        out_specs=pl.BlockSpec((blk, W128), lambda j: (j, 0)),
        out_shape=jax.ShapeDtypeStruct((NP, W128), _f32),
    )(d0, d1, h1pre)


def _tc2_body(d0, d1, a0, a1, w2, b1, h2s):
    dis = _dis_from(d0[...], d1[...])
    agg = a0[:, 0:GE] + a1[:, 0:GE]
    out1 = jnp.maximum(agg * dis + b1[...], 0.0)
    h2s[...] = dis * jnp.dot(out1, w2[...], preferred_element_type=_f32)


def _tc2(d0, d1, a0, a1, w2, b1):
    blk = NP // 80
    return pl.pallas_call(
        _tc2_body,
        grid=(80,),
        in_specs=[
            pl.BlockSpec((blk, W128), lambda j: (j, 0)),
            pl.BlockSpec((blk, W128), lambda j: (j, 0)),
            pl.BlockSpec((blk, W128), lambda j: (j, 0)),
            pl.BlockSpec((blk, W128), lambda j: (j, 0)),
            pl.BlockSpec((GE, W128), lambda j: (0, 0)),
            pl.BlockSpec((1, GE), lambda j: (0, 0)),
        ],
        out_specs=pl.BlockSpec((blk, W128), lambda j: (j, 0)),
        out_shape=jax.ShapeDtypeStruct((NP, W128), _f32),
    )(d0, d1, a0, a1, w2, b1)


def _tc3_body(d0, d1, q0, q1, b2, gid, mu, lv, acc):
    j = pl.program_id(0)
    dis = _dis_from(d0[...], d1[...])
    ne = (q0[...] + q1[...]) * dis + b2[...]
    g = lax.broadcasted_iota(_i32, (NG, NP // 80), 0)
    sel = (g == gid[0]).astype(_f32)

    @pl.when(j == 0)
    def _():
        acc[...] = jnp.zeros_like(acc)

    acc[...] += jnp.dot(sel, ne, preferred_element_type=_f32)

    @pl.when(j == pl.num_programs(0) - 1)
    def _():
        mu[...] = acc[:, 0:GE]
        lv[...] = jnp.tanh(acc[:, GE:W128])


def _tc3(d0, d1, q0, q1, b2, gid):
    blk = NP // 80
    return pl.pallas_call(
        _tc3_body,
        grid=(80,),
        in_specs=[
            pl.BlockSpec((blk, W128), lambda j: (j, 0)),
            pl.BlockSpec((blk, W128), lambda j: (j, 0)),
            pl.BlockSpec((blk, W128), lambda j: (j, 0)),
            pl.BlockSpec((blk, W128), lambda j: (j, 0)),
            pl.BlockSpec((1, W128), lambda j: (0, 0)),
            pl.BlockSpec((1, 1, blk), lambda j: (j, 0, 0)),
        ],
        out_specs=[
            pl.BlockSpec((NG, GE), lambda j: (0, 0)),
            pl.BlockSpec((NG, GE), lambda j: (0, 0)),
        ],
        out_shape=[
            jax.ShapeDtypeStruct((NG, GE), _f32),
            jax.ShapeDtypeStruct((NG, GE), _f32),
        ],
        scratch_shapes=[pltpu.VMEM((NG, W128), _f32)],
    )(d0, d1, q0, q1, b2, gid)


def kernel(operation_id, service_id, status_id, node_depth, edge_index,
           graph_ids, op_table, svc_table, st_table, depth_table,
           W1, b1, W2, b2):
    del node_depth, depth_table
    e0 = edge_index[0].astype(_i32)
    e1 = edge_index[1].astype(_i32)
    epad = PE + NW * CH - ED
    src_all = jnp.pad(jnp.concatenate([e0, e1]), (0, epad),
                      constant_values=EPAD)
    dst_all = jnp.pad(jnp.concatenate([e1, e0]), (0, epad),
                      constant_values=EPAD)

    pad = NP - N
    opi = jnp.pad(operation_id.astype(_i32), (0, pad))
    svci = jnp.pad(service_id.astype(_i32), (0, pad))
    sti = jnp.pad(status_id.astype(_i32), (0, pad))
    gid = jnp.pad(graph_ids.astype(_i32), (0, pad),
                  constant_values=NG).reshape(80, 1, NP // 80)

    st_p = jnp.pad(st_table, ((0, 6), (0, 0)))
    ones128 = jnp.ones((CH, W128), _f32)
    z128 = jnp.zeros((RBL, W128), _f32)

    opw, svcw, stw = _tc0(op_table, svc_table, st_p, W1)

    degf, = _sc_deg(dst_all, ones128, z128)
    d0, d1 = degf[:NP], degf[NP:]

    h1pre, = _sc_embed(opi, svci, sti, opw, svcw, stw)

    h1s = _tc1(d0, d1, h1pre)

    agg1, = _sc_agg(src_all, dst_all, h1s, z128)

    h2s = _tc2(d0, d1, agg1[:NP], agg1[NP:], W2, b1.reshape(1, GE))

    agg2, = _sc_agg(src_all, dst_all, h2s, z128)

    mu, lv = _tc3(d0, d1, agg2[:NP], agg2[NP:], b2.reshape(1, 2 * GE), gid)
    return (mu, lv)


# speedup vs baseline: 1.0951x; 1.0951x over previous
import functools

import jax
import jax.numpy as jnp
from jax import lax
from jax.experimental import pallas as pl
from jax.experimental.pallas import tpu as pltpu
from jax.experimental.pallas import tpu_sc as plsc

N = 10000
E = 320000
ED = 2 * E
EMB = 32
GE = 64
W128 = 2 * GE
NG = 128

NC = 2
NS = 16
NW = NC * NS

NP = 10240
NPW = NP // NW
NBC = 4
NBW = NPW // NBC
CH = 128
NCH = 160
CHE = 64
NCHE = 320
RING = 4
PE = NW * NCH * CH
EPAD = NP - 1
RPT = NP // NS
RBL = 128
RB = RPT // RBL

_f32 = jnp.float32
_i32 = jnp.int32

_MESH = plsc.VectorSubcoreMesh(core_axis_name="c", subcore_axis_name="s")


def _tc0_body(op_t, svc_t, st_t, w1, opw, svcw, stw):
    w = w1[...]
    z = jnp.zeros((1, GE), _f32)
    opw[:, 0:GE] = jnp.dot(op_t[...], w[0:EMB, :], preferred_element_type=_f32)
    opw[:, GE:W128] = jnp.broadcast_to(z, (1000, GE))
    svcw[:, 0:GE] = jnp.dot(svc_t[...], w[EMB:2 * EMB, :],
                            preferred_element_type=_f32)
    svcw[:, GE:W128] = jnp.broadcast_to(z, (200, GE))
    stw[:, 0:GE] = jnp.dot(st_t[...], w[2 * EMB:3 * EMB, :],
                           preferred_element_type=_f32)
    stw[:, GE:W128] = jnp.broadcast_to(z, (16, GE))


def _tc0(op_t, svc_t, st_t, w1):
    return pl.pallas_call(
        _tc0_body,
        out_shape=[
            jax.ShapeDtypeStruct((1000, W128), _f32),
            jax.ShapeDtypeStruct((200, W128), _f32),
            jax.ShapeDtypeStruct((16, W128), _f32),
        ],
    )(op_t, svc_t, st_t, w1)


@functools.partial(
    pl.kernel,
    mesh=_MESH,
    out_type=[jax.ShapeDtypeStruct((NC * NP, W128), _f32)],
    scratch_types=[
        pltpu.VMEM((CH,), _i32),
        pltpu.VMEM((CH,), _i32),
        pltpu.VMEM((CH, W128), _f32),
        pltpu.VMEM((RBL, W128), _f32),
        pltpu.VMEM_SHARED((NP, W128), _f32),
        pltpu.SemaphoreType.DMA,
        pltpu.SemaphoreType.DMA,
    ],
)
def _sc_deg(dst_hbm, ones_hbm, z_hbm, deg_out, dstia, dstib, onesb, zb,
            acc_sp, sema, semb):
    c = lax.axis_index("c")
    s = lax.axis_index("s")
    wid = c * NS + s

    pltpu.sync_copy(z_hbm, zb)
    for p in range(RB):
        pltpu.sync_copy(zb, acc_sp.at[pl.ds(s * RPT + p * RBL, RBL)])
    pltpu.sync_copy(ones_hbm, onesb)
    plsc.subcore_barrier()

    def body(m, carry):
        base = (2 * m * NW + wid) * CH
        pltpu.sync_copy(dst_hbm.at[pl.ds(base, CH)], dstia)
        pltpu.async_copy(onesb, acc_sp.at[dstia], sema, add=True)
        pltpu.sync_copy(dst_hbm.at[pl.ds(base + NW * CH, CH)], dstib)
        pltpu.async_copy(onesb, acc_sp.at[dstib], semb, add=True)
        pltpu.make_async_copy(onesb, acc_sp.at[dstia], sema).wait()
        pltpu.make_async_copy(onesb, acc_sp.at[dstib], semb).wait()
        return carry

    lax.fori_loop(0, NCH // 2, body, 0)
    plsc.subcore_barrier()

    for p in range(RB):
        off = s * RPT + p * RBL
        pltpu.sync_copy(acc_sp.at[pl.ds(off, RBL)], zb)
        pltpu.sync_copy(zb, deg_out.at[pl.ds(c * NP + off, RBL)])


@functools.partial(
    pl.kernel,
    mesh=_MESH,
    out_type=[jax.ShapeDtypeStruct((NP, W128), _f32)],
    scratch_types=[
        pltpu.VMEM((NBW,), _i32),
        pltpu.VMEM((NBW,), _i32),
        pltpu.VMEM((NBW,), _i32),
        pltpu.VMEM((NBW, W128), _f32),
        pltpu.VMEM((NBW, W128), _f32),
        pltpu.VMEM((NBW, W128), _f32),
    ],
)
def _sc_embed(opi_hbm, svci_hbm, sti_hbm, opw_hbm, svcw_hbm, stw_hbm,
              h1pre_out, opi, svci, sti, bufa, bufb, bufc):
    c = lax.axis_index("c")
    s = lax.axis_index("s")
    wid = c * NS + s

    for k in range(NBC):
        off = wid * NPW + k * NBW
        sl = pl.ds(off, NBW)
        pltpu.sync_copy(opi_hbm.at[sl], opi)
        pltpu.sync_copy(svci_hbm.at[sl], svci)
        pltpu.sync_copy(sti_hbm.at[sl], sti)
        pltpu.sync_copy(opw_hbm.at[opi], bufa)
        pltpu.sync_copy(svcw_hbm.at[svci], bufb)
        pltpu.sync_copy(stw_hbm.at[sti], bufc)

        def add_body(r, carry):
            for q in range(W128 // 16):
                cs = pl.ds(q * 16, 16)
                bufa[r, cs] = bufa[r, cs] + bufb[r, cs] + bufc[r, cs]
            return carry

        lax.fori_loop(0, NBW, add_body, 0)
        pltpu.sync_copy(bufa, h1pre_out.at[sl])


@functools.partial(
    pl.kernel,
    mesh=_MESH,
    out_type=[jax.ShapeDtypeStruct((NC * NP, W128), _f32)],
    scratch_types=(
        [pltpu.VMEM((CHE,), _i32) for _ in range(RING)]
        + [pltpu.VMEM((CHE,), _i32) for _ in range(RING)]
        + [pltpu.VMEM((CHE, W128), _f32) for _ in range(RING)]
        + [pltpu.VMEM_SHARED((NP, W128), _f32)]
        + [pltpu.SemaphoreType.DMA for _ in range(RING)]
    ),
)
def _sc_agg(src_hbm, dst_hbm, h_hbm, z_hbm, agg_out, *refs):
    srci = refs[0:RING]
    dsti = refs[RING:2 * RING]
    rows = refs[2 * RING:3 * RING]
    acc_sp = refs[3 * RING]
    sem = refs[3 * RING + 1:]

    c = lax.axis_index("c")
    s = lax.axis_index("s")
    wid = c * NS + s
    tb = wid * NCHE * CHE

    pltpu.sync_copy(z_hbm, rows[0])
    for p in range(RPT // CHE):
        pltpu.sync_copy(rows[0], acc_sp.at[pl.ds(s * RPT + p * CHE, CHE)])
    plsc.subcore_barrier()

    for r in range(RING):
        off = tb + r * CHE
        pltpu.sync_copy(src_hbm.at[pl.ds(off, CHE)], srci[r])
        pltpu.sync_copy(dst_hbm.at[pl.ds(off, CHE)], dsti[r])
        pltpu.async_copy(h_hbm.at[srci[r]], rows[r], sem[r])

    def body(m, carry):
        for r in range(RING):
            pltpu.make_async_copy(h_hbm.at[srci[r]], rows[r], sem[r]).wait()
            pltpu.sync_copy(rows[r], acc_sp.at[dsti[r]], add=True)
            off = tb + (RING * m + r + RING) * CHE
            pltpu.sync_copy(src_hbm.at[pl.ds(off, CHE)], srci[r])
            pltpu.sync_copy(dst_hbm.at[pl.ds(off, CHE)], dsti[r])
            pltpu.async_copy(h_hbm.at[srci[r]], rows[r], sem[r])
        return carry

    lax.fori_loop(0, NCHE // RING - 1, body, 0)
    for r in range(RING):
        pltpu.make_async_copy(h_hbm.at[srci[r]], rows[r], sem[r]).wait()
        pltpu.sync_copy(rows[r], acc_sp.at[dsti[r]], add=True)
    plsc.subcore_barrier()

    for p in range(RPT // CHE):
        off = s * RPT + p * CHE
        pltpu.sync_copy(acc_sp.at[pl.ds(off, CHE)], rows[0])
        pltpu.sync_copy(rows[0], agg_out.at[pl.ds(c * NP + off, CHE)])


def _dis_from(d0, d1):
    deg = d0[:, 0:1] + d1[:, 0:1]
    return jnp.where(deg > 0.0, lax.rsqrt(jnp.maximum(deg, 1.0)), 0.0)


def _tc1_body(d0, d1, h1pre, h1s):
    h1s[...] = h1pre[...] * _dis_from(d0[...], d1[...])


def _tc1(d0, d1, h1pre):
    blk = NP // 80
    return pl.pallas_call(
        _tc1_body,
        grid=(80,),
        in_specs=[
            pl.BlockSpec((blk, W128), lambda j: (j, 0)),
            pl.BlockSpec((blk, W128), lambda j: (j, 0)),
            pl.BlockSpec((blk, W128), lambda j: (j, 0)),
        ],
        out_specs=pl.BlockSpec((blk, W128), lambda j: (j, 0)),
        out_shape=jax.ShapeDtypeStruct((NP, W128), _f32),
    )(d0, d1, h1pre)


def _tc2_body(d0, d1, a0, a1, w2, b1, h2s):
    dis = _dis_from(d0[...], d1[...])
    agg = a0[:, 0:GE] + a1[:, 0:GE]
    out1 = jnp.maximum(agg * dis + b1[...], 0.0)
    h2s[...] = dis * jnp.dot(out1, w2[...], preferred_element_type=_f32)


def _tc2(d0, d1, a0, a1, w2, b1):
    blk = NP // 80
    return pl.pallas_call(
        _tc2_body,
        grid=(80,),
        in_specs=[
            pl.BlockSpec((blk, W128), lambda j: (j, 0)),
            pl.BlockSpec((blk, W128), lambda j: (j, 0)),
            pl.BlockSpec((blk, W128), lambda j: (j, 0)),
            pl.BlockSpec((blk, W128), lambda j: (j, 0)),
            pl.BlockSpec((GE, W128), lambda j: (0, 0)),
            pl.BlockSpec((1, GE), lambda j: (0, 0)),
        ],
        out_specs=pl.BlockSpec((blk, W128), lambda j: (j, 0)),
        out_shape=jax.ShapeDtypeStruct((NP, W128), _f32),
    )(d0, d1, a0, a1, w2, b1)


def _tc3_body(d0, d1, q0, q1, b2, gid, mu, lv, acc):
    j = pl.program_id(0)
    dis = _dis_from(d0[...], d1[...])
    ne = (q0[...] + q1[...]) * dis + b2[...]
    g = lax.broadcasted_iota(_i32, (NG, NP // 80), 0)
    sel = (g == gid[0]).astype(_f32)

    @pl.when(j == 0)
    def _():
        acc[...] = jnp.zeros_like(acc)

    acc[...] += jnp.dot(sel, ne, preferred_element_type=_f32)

    @pl.when(j == pl.num_programs(0) - 1)
    def _():
        mu[...] = acc[:, 0:GE]
        lv[...] = jnp.tanh(acc[:, GE:W128])


def _tc3(d0, d1, q0, q1, b2, gid):
    blk = NP // 80
    return pl.pallas_call(
        _tc3_body,
        grid=(80,),
        in_specs=[
            pl.BlockSpec((blk, W128), lambda j: (j, 0)),
            pl.BlockSpec((blk, W128), lambda j: (j, 0)),
            pl.BlockSpec((blk, W128), lambda j: (j, 0)),
            pl.BlockSpec((blk, W128), lambda j: (j, 0)),
            pl.BlockSpec((1, W128), lambda j: (0, 0)),
            pl.BlockSpec((1, 1, blk), lambda j: (j, 0, 0)),
        ],
        out_specs=[
            pl.BlockSpec((NG, GE), lambda j: (0, 0)),
            pl.BlockSpec((NG, GE), lambda j: (0, 0)),
        ],
        out_shape=[
            jax.ShapeDtypeStruct((NG, GE), _f32),
            jax.ShapeDtypeStruct((NG, GE), _f32),
        ],
        scratch_shapes=[pltpu.VMEM((NG, W128), _f32)],
    )(d0, d1, q0, q1, b2, gid)


def kernel(operation_id, service_id, status_id, node_depth, edge_index,
           graph_ids, op_table, svc_table, st_table, depth_table,
           W1, b1, W2, b2):
    del node_depth, depth_table
    e0 = edge_index[0].astype(_i32)
    e1 = edge_index[1].astype(_i32)
    epad = PE + NW * CH - ED
    src_all = jnp.pad(jnp.concatenate([e0, e1]), (0, epad),
                      constant_values=EPAD)
    dst_all = jnp.pad(jnp.concatenate([e1, e0]), (0, epad),
                      constant_values=EPAD)

    pad = NP - N
    opi = jnp.pad(operation_id.astype(_i32), (0, pad))
    svci = jnp.pad(service_id.astype(_i32), (0, pad))
    sti = jnp.pad(status_id.astype(_i32), (0, pad))
    gid = jnp.pad(graph_ids.astype(_i32), (0, pad),
                  constant_values=NG).reshape(80, 1, NP // 80)

    st_p = jnp.pad(st_table, ((0, 6), (0, 0)))
    ones128 = jnp.ones((CH, W128), _f32)
    z128 = jnp.zeros((RBL, W128), _f32)
    z64 = jnp.zeros((CHE, W128), _f32)

    opw, svcw, stw = _tc0(op_table, svc_table, st_p, W1)

    degf, = _sc_deg(dst_all, ones128, z128)
    d0, d1 = degf[:NP], degf[NP:]

    h1pre, = _sc_embed(opi, svci, sti, opw, svcw, stw)

    h1s = _tc1(d0, d1, h1pre)

    agg1, = _sc_agg(src_all, dst_all, h1s, z64)

    h2s = _tc2(d0, d1, agg1[:NP], agg1[NP:], W2, b1.reshape(1, GE))

    agg2, = _sc_agg(src_all, dst_all, h2s, z64)

    mu, lv = _tc3(d0, d1, agg2[:NP], agg2[NP:], b2.reshape(1, 2 * GE), gid)
    return (mu, lv)
